# baseline (speedup 1.0000x reference)
import functools

import jax
import jax.numpy as jnp
from jax import lax
from jax.experimental import pallas as pl
from jax.experimental.pallas import tpu as pltpu
from jax.experimental.pallas import tpu_sc as plsc

NC, NS, L = 2, 16, 16
NW = NC * NS
N = 2048
B = 8
HALF = N // 2
NBLK = N // L
K = 16


def _round_bf16(v):
    u = plsc.bitcast(v, jnp.uint32)
    r = (u + jnp.uint32(0x7FFF) + ((u >> jnp.uint32(16)) & jnp.uint32(1)))
    r = r & jnp.uint32(0xFFFF0000)
    return plsc.bitcast(r, jnp.float32)


def _knn_body(pts_hbm, out_hbm, cand_v, candr_v, cc_v, top_v, thr_v, acc_v):
    wid = lax.axis_index("s") * NC + lax.axis_index("c")
    prob = wid // 2
    half = wid % 2
    pltpu.sync_copy(pts_hbm.at[prob], cand_v)

    inf_v = jnp.full((L,), jnp.inf, dtype=jnp.float32)
    zero_v = jnp.zeros((L,), dtype=jnp.float32)

    def pre_body(j, carry):
        base = j * L
        cx = cand_v[0, pl.ds(base, L)]
        cy = cand_v[1, pl.ds(base, L)]
        cz = cand_v[2, pl.ds(base, L)]
        candr_v[0, pl.ds(base, L)] = _round_bf16(cx) * 2.0
        candr_v[1, pl.ds(base, L)] = _round_bf16(cy) * 2.0
        candr_v[2, pl.ds(base, L)] = _round_bf16(cz) * 2.0
        cc_v[pl.ds(base, L)] = (cx * cx + cy * cy) + cz * cz
        return carry

    lax.fori_loop(0, NBLK, pre_body, 0)

    def qblk_body(qb, acc_outer):
        qbase = half * HALF + qb * L
        qxb = cand_v[0, pl.ds(qbase, L)]
        qyb = cand_v[1, pl.ds(qbase, L)]
        qzb = cand_v[2, pl.ds(qbase, L)]
        qqb = (qxb * qxb + qyb * qyb) + qzb * qzb
        qxrb = _round_bf16(qxb)
        qyrb = _round_bf16(qyb)
        qzrb = _round_bf16(qzb)

        def make_row(lane):
            qq = jnp.full((L,), qqb[lane], dtype=jnp.float32)
            qx = jnp.full((L,), qxrb[lane], dtype=jnp.float32)
            qy = jnp.full((L,), qyrb[lane], dtype=jnp.float32)
            qz = jnp.full((L,), qzrb[lane], dtype=jnp.float32)
            top_v[...] = inf_v
            thr_v[...] = inf_v

            def blk_body(j, carry):
                base = j * L
                inner2 = (candr_v[0, pl.ds(base, L)] * qx
                          + candr_v[1, pl.ds(base, L)] * qy)
                inner2 = inner2 + candr_v[2, pl.ds(base, L)] * qz
                d = (qq - inner2) + cc_v[pl.ds(base, L)]
                nbeat = plsc.all_reduce_population_count(d < thr_v[...])

                @pl.when(nbeat[0] > 0)
                def _merge():
                    dcl = jnp.maximum(d, zero_v)
                    c_desc, _ = plsc.sort_key_val(dcl, dcl, descending=True)
                    lo = jnp.minimum(top_v[...], c_desc)
                    top_n, _ = plsc.sort_key_val(lo, lo)
                    top_v[...] = top_n
                    thr_v[...] = jnp.full((L,), top_n[L - 1], dtype=jnp.float32)

                return carry

            lax.fori_loop(0, NBLK, blk_body, 0)
            return top_v[...]

        acc = acc_outer
        for lane in range(L):
            acc = acc + make_row(lane)
        return acc

    acc = lax.fori_loop(0, HALF // L, qblk_body,
                        jnp.zeros((L,), dtype=jnp.float32))
    acc_v[...] = acc
    pltpu.sync_copy(acc_v, out_hbm.at[wid])


_knn = functools.partial(
    pl.kernel,
    out_type=jax.ShapeDtypeStruct((NW, L), jnp.float32),
    mesh=plsc.VectorSubcoreMesh(core_axis_name="c", subcore_axis_name="s",
                                num_cores=NC, num_subcores=NS),
    scratch_types=[
        pltpu.VMEM((3, N), jnp.float32),
        pltpu.VMEM((3, N), jnp.float32),
        pltpu.VMEM((N,), jnp.float32),
        pltpu.VMEM((L,), jnp.float32),
        pltpu.VMEM((L,), jnp.float32),
        pltpu.VMEM((L,), jnp.float32),
    ],
    compiler_params=pltpu.CompilerParams(needs_layout_passes=False),
)(_knn_body)


def kernel(seed, gt_s):
    pts = jnp.stack([seed, gt_s])
    pts = pts.transpose(0, 1, 3, 2).reshape(2 * B, 3, N)
    out = _knn(pts)
    per_prob = out.sum(axis=1).reshape(2 * B, 2).sum(axis=1)
    means = (per_prob / (N * K)).reshape(2, B)
    return jnp.mean((means[0] - means[1]) ** 2)


# speedup vs baseline: 1.8805x; 1.8805x over previous
import functools

import jax
import jax.numpy as jnp
from jax import lax
from jax.experimental import pallas as pl
from jax.experimental.pallas import tpu as pltpu
from jax.experimental.pallas import tpu_sc as plsc

NC, NS, L = 2, 16, 16
NW = NC * NS
N = 2048
B = 8
HALF = N // 2
NBLK = N // L
K = 16


def _round_bf16(v):
    u = plsc.bitcast(v, jnp.uint32)
    r = (u + jnp.uint32(0x7FFF) + ((u >> jnp.uint32(16)) & jnp.uint32(1)))
    r = r & jnp.uint32(0xFFFF0000)
    return plsc.bitcast(r, jnp.float32)


BIG = 3.0e38


def _knn_body(pts_hbm, out_hbm, cand_v, candr_v, cc_v, drow_v, buf_v, acc_v):
    wid = lax.axis_index("s") * NC + lax.axis_index("c")
    prob = wid // 2
    half = wid % 2
    pltpu.sync_copy(pts_hbm.at[prob], cand_v)

    big_v = jnp.full((L,), BIG, dtype=jnp.float32)
    zero_v = jnp.zeros((L,), dtype=jnp.float32)

    def pre_body(j, carry):
        base = j * L
        cx = cand_v[0, pl.ds(base, L)]
        cy = cand_v[1, pl.ds(base, L)]
        cz = cand_v[2, pl.ds(base, L)]
        candr_v[0, pl.ds(base, L)] = _round_bf16(cx) * 2.0
        candr_v[1, pl.ds(base, L)] = _round_bf16(cy) * 2.0
        candr_v[2, pl.ds(base, L)] = _round_bf16(cz) * 2.0
        cc_v[pl.ds(base, L)] = (cx * cx + cy * cy) + cz * cz
        return carry

    lax.fori_loop(0, NBLK, pre_body, 0)

    def qblk_body(qb, acc_outer):
        qbase = half * HALF + qb * L
        qxb = cand_v[0, pl.ds(qbase, L)]
        qyb = cand_v[1, pl.ds(qbase, L)]
        qzb = cand_v[2, pl.ds(qbase, L)]
        qqb = (qxb * qxb + qyb * qyb) + qzb * qzb
        qxrb = _round_bf16(qxb)
        qyrb = _round_bf16(qyb)
        qzrb = _round_bf16(qzb)

        def make_row(lane):
            qq = jnp.full((L,), qqb[lane], dtype=jnp.float32)
            qx = jnp.full((L,), qxrb[lane], dtype=jnp.float32)
            qy = jnp.full((L,), qyrb[lane], dtype=jnp.float32)
            qz = jnp.full((L,), qzrb[lane], dtype=jnp.float32)

            def pass_a(j, w):
                base = j * L
                inner2 = (candr_v[0, pl.ds(base, L)] * qx
                          + candr_v[1, pl.ds(base, L)] * qy)
                inner2 = inner2 + candr_v[2, pl.ds(base, L)] * qz
                d = (qq - inner2) + cc_v[pl.ds(base, L)]
                drow_v[pl.ds(base, L)] = d
                return jnp.minimum(w, d)

            w = lax.fori_loop(0, NBLK, pass_a, big_v)
            ws, _ = plsc.sort_key_val(w, w)
            thr = jnp.full((L,), ws[L - 1], dtype=jnp.float32)

            def pass_b(j, cnt):
                d = drow_v[pl.ds(j * L, L)]
                m = d <= thr
                plsc.store_compressed(buf_v.at[pl.ds(cnt, L)], d, mask=m)
                return cnt + plsc.all_reduce_population_count(m)[0]

            cnt = lax.fori_loop(0, NBLK, pass_b, jnp.int32(0))
            buf_v[pl.ds(cnt, L)] = big_v

            nmerge = lax.div(cnt + (L - 1), jnp.int32(L))

            def pass_c(t, top):
                blk = buf_v[pl.ds(t * L, L)]
                dcl = jnp.maximum(blk, zero_v)
                c_desc, _ = plsc.sort_key_val(dcl, dcl, descending=True)
                lo = jnp.minimum(top, c_desc)
                top_n, _ = plsc.sort_key_val(lo, lo)
                return top_n

            return lax.fori_loop(0, nmerge, pass_c, big_v)

        acc = acc_outer
        for lane in range(L):
            acc = acc + make_row(lane)
        return acc

    acc = lax.fori_loop(0, HALF // L, qblk_body,
                        jnp.zeros((L,), dtype=jnp.float32))
    acc_v[...] = acc
    pltpu.sync_copy(acc_v, out_hbm.at[wid])


_knn = functools.partial(
    pl.kernel,
    out_type=jax.ShapeDtypeStruct((NW, L), jnp.float32),
    mesh=plsc.VectorSubcoreMesh(core_axis_name="c", subcore_axis_name="s",
                                num_cores=NC, num_subcores=NS),
    scratch_types=[
        pltpu.VMEM((3, N), jnp.float32),
        pltpu.VMEM((3, N), jnp.float32),
        pltpu.VMEM((N,), jnp.float32),
        pltpu.VMEM((N,), jnp.float32),
        pltpu.VMEM((N + L,), jnp.float32),
        pltpu.VMEM((L,), jnp.float32),
    ],
    compiler_params=pltpu.CompilerParams(needs_layout_passes=False),
)(_knn_body)


def kernel(seed, gt_s):
    pts = jnp.stack([seed, gt_s])
    pts = pts.transpose(0, 1, 3, 2).reshape(2 * B, 3, N)
    out = _knn(pts)
    per_prob = out.sum(axis=1).reshape(2 * B, 2).sum(axis=1)
    means = (per_prob / (N * K)).reshape(2, B)
    return jnp.mean((means[0] - means[1]) ** 2)


# speedup vs baseline: 2.6040x; 1.3848x over previous
import functools

import jax
import jax.numpy as jnp
from jax import lax
from jax.experimental import pallas as pl
from jax.experimental.pallas import tpu as pltpu
from jax.experimental.pallas import tpu_sc as plsc

NC, NS, L = 2, 16, 16
NW = NC * NS
N = 2048
B = 8
HALF = N // 2
NBLK = N // L
K = 16


def _round_bf16(v):
    u = plsc.bitcast(v, jnp.uint32)
    r = (u + jnp.uint32(0x7FFF) + ((u >> jnp.uint32(16)) & jnp.uint32(1)))
    r = r & jnp.uint32(0xFFFF0000)
    return plsc.bitcast(r, jnp.float32)


BIG = 3.0e38


def _knn_body(pts_hbm, out_hbm, cand_v, candr_v, cc_v, drow_v, drow1_v,
              buf_v, acc_v):
    wid = lax.axis_index("s") * NC + lax.axis_index("c")
    prob = wid // 2
    half = wid % 2
    pltpu.sync_copy(pts_hbm.at[prob], cand_v)

    big_v = jnp.full((L,), BIG, dtype=jnp.float32)
    zero_v = jnp.zeros((L,), dtype=jnp.float32)

    def pre_body(j, carry):
        base = j * L
        cx = cand_v[0, pl.ds(base, L)]
        cy = cand_v[1, pl.ds(base, L)]
        cz = cand_v[2, pl.ds(base, L)]
        candr_v[0, pl.ds(base, L)] = _round_bf16(cx) * 2.0
        candr_v[1, pl.ds(base, L)] = _round_bf16(cy) * 2.0
        candr_v[2, pl.ds(base, L)] = _round_bf16(cz) * 2.0
        cc_v[pl.ds(base, L)] = (cx * cx + cy * cy) + cz * cz
        return carry

    lax.fori_loop(0, NBLK, pre_body, 0)

    def qblk_body(qb, acc_outer):
        qbase = half * HALF + qb * L
        qxb = cand_v[0, pl.ds(qbase, L)]
        qyb = cand_v[1, pl.ds(qbase, L)]
        qzb = cand_v[2, pl.ds(qbase, L)]
        qqb = (qxb * qxb + qyb * qyb) + qzb * qzb
        qxrb = _round_bf16(qxb)
        qyrb = _round_bf16(qyb)
        qzrb = _round_bf16(qzb)

        def splat(vec, lane):
            return jnp.full((L,), vec[lane], dtype=jnp.float32)

        def finish_row(drow, w):
            ws, _ = plsc.sort_key_val(w, w)
            thr = jnp.full((L,), ws[L - 1], dtype=jnp.float32)

            def pass_b(j, cnt):
                for u in range(2):
                    d = drow[pl.ds((2 * j + u) * L, L)]
                    m = d <= thr
                    plsc.store_compressed(buf_v.at[pl.ds(cnt, L)], d, mask=m)
                    cnt = cnt + plsc.all_reduce_population_count(m)[0]
                return cnt

            cnt = lax.fori_loop(0, NBLK // 2, pass_b, jnp.int32(0))
            buf_v[pl.ds(cnt, L)] = big_v

            nmerge = lax.div(cnt + (L - 1), jnp.int32(L))

            def pass_c(t, top):
                blk = buf_v[pl.ds(t * L, L)]
                dcl = jnp.maximum(blk, zero_v)
                c_desc, _ = plsc.sort_key_val(dcl, dcl, descending=True)
                lo = jnp.minimum(top, c_desc)
                top_n, _ = plsc.sort_key_val(lo, lo)
                return top_n

            return lax.fori_loop(0, nmerge, pass_c, big_v)

        acc = acc_outer
        for pair in range(L // 2):
            l0, l1 = 2 * pair, 2 * pair + 1
            qq0, qx0, qy0, qz0 = (splat(qqb, l0), splat(qxrb, l0),
                                  splat(qyrb, l0), splat(qzrb, l0))
            qq1, qx1, qy1, qz1 = (splat(qqb, l1), splat(qxrb, l1),
                                  splat(qyrb, l1), splat(qzrb, l1))

            def pass_a(j, carry):
                w0, w1 = carry
                for u in range(2):
                    base = (2 * j + u) * L
                    c2x = candr_v[0, pl.ds(base, L)]
                    c2y = candr_v[1, pl.ds(base, L)]
                    c2z = candr_v[2, pl.ds(base, L)]
                    cc = cc_v[pl.ds(base, L)]
                    i0 = (c2x * qx0 + c2y * qy0) + c2z * qz0
                    d0 = (qq0 - i0) + cc
                    drow_v[pl.ds(base, L)] = d0
                    w0 = jnp.minimum(w0, d0)
                    i1 = (c2x * qx1 + c2y * qy1) + c2z * qz1
                    d1 = (qq1 - i1) + cc
                    drow1_v[pl.ds(base, L)] = d1
                    w1 = jnp.minimum(w1, d1)
                return w0, w1

            w0, w1 = lax.fori_loop(0, NBLK // 2, pass_a, (big_v, big_v))
            acc = acc + finish_row(drow_v, w0)
            acc = acc + finish_row(drow1_v, w1)
        return acc

    acc = lax.fori_loop(0, HALF // L, qblk_body,
                        jnp.zeros((L,), dtype=jnp.float32))
    acc_v[...] = acc
    pltpu.sync_copy(acc_v, out_hbm.at[wid])


_knn = functools.partial(
    pl.kernel,
    out_type=jax.ShapeDtypeStruct((NW, L), jnp.float32),
    mesh=plsc.VectorSubcoreMesh(core_axis_name="c", subcore_axis_name="s",
                                num_cores=NC, num_subcores=NS),
    scratch_types=[
        pltpu.VMEM((3, N), jnp.float32),
        pltpu.VMEM((3, N), jnp.float32),
        pltpu.VMEM((N,), jnp.float32),
        pltpu.VMEM((N,), jnp.float32),
        pltpu.VMEM((N,), jnp.float32),
        pltpu.VMEM((N + L,), jnp.float32),
        pltpu.VMEM((L,), jnp.float32),
    ],
    compiler_params=pltpu.CompilerParams(needs_layout_passes=False),
)(_knn_body)


def kernel(seed, gt_s):
    pts = jnp.stack([seed, gt_s])
    pts = pts.transpose(0, 1, 3, 2).reshape(2 * B, 3, N)
    out = _knn(pts)
    per_prob = out.sum(axis=1).reshape(2 * B, 2).sum(axis=1)
    means = (per_prob / (N * K)).reshape(2, B)
    return jnp.mean((means[0] - means[1]) ** 2)


# speedup vs baseline: 3.1230x; 1.1993x over previous
import functools

import jax
import jax.numpy as jnp
from jax import lax
from jax.experimental import pallas as pl
from jax.experimental.pallas import tpu as pltpu
from jax.experimental.pallas import tpu_sc as plsc

NC, NS, L = 2, 16, 16
NW = NC * NS
N = 2048
B = 8
HALF = N // 2
NBLK = N // L
K = 16


def _round_bf16(v):
    u = plsc.bitcast(v, jnp.uint32)
    r = (u + jnp.uint32(0x7FFF) + ((u >> jnp.uint32(16)) & jnp.uint32(1)))
    r = r & jnp.uint32(0xFFFF0000)
    return plsc.bitcast(r, jnp.float32)


BIG = 3.0e38


def _knn_body(pts_hbm, out_hbm, cand_v, candr_v, cc_v, drow_v, drow1_v,
              buf_v, buf1_v, acc_v):
    wid = lax.axis_index("s") * NC + lax.axis_index("c")
    prob = wid // 2
    half = wid % 2
    pltpu.sync_copy(pts_hbm.at[prob], cand_v)

    big_v = jnp.full((L,), BIG, dtype=jnp.float32)
    zero_v = jnp.zeros((L,), dtype=jnp.float32)

    def pre_body(j, carry):
        base = j * L
        cx = cand_v[0, pl.ds(base, L)]
        cy = cand_v[1, pl.ds(base, L)]
        cz = cand_v[2, pl.ds(base, L)]
        candr_v[0, pl.ds(base, L)] = _round_bf16(cx) * 2.0
        candr_v[1, pl.ds(base, L)] = _round_bf16(cy) * 2.0
        candr_v[2, pl.ds(base, L)] = _round_bf16(cz) * 2.0
        cc_v[pl.ds(base, L)] = (cx * cx + cy * cy) + cz * cz
        return carry

    lax.fori_loop(0, NBLK, pre_body, 0)

    def qblk_body(qb, acc_outer):
        qbase = half * HALF + qb * L
        qxb = cand_v[0, pl.ds(qbase, L)]
        qyb = cand_v[1, pl.ds(qbase, L)]
        qzb = cand_v[2, pl.ds(qbase, L)]
        qqb = (qxb * qxb + qyb * qyb) + qzb * qzb
        qxrb = _round_bf16(qxb)
        qyrb = _round_bf16(qyb)
        qzrb = _round_bf16(qzb)

        def splat(vec, lane):
            return jnp.full((L,), vec[lane], dtype=jnp.float32)

        def thr_of(w):
            ws, _ = plsc.sort_key_val(w, w)
            return jnp.full((L,), ws[L - 1], dtype=jnp.float32)

        def pass_c(buf, cnt):
            buf[pl.ds(cnt, L)] = big_v
            nmerge = lax.div(cnt + (L - 1), jnp.int32(L))

            def body(t, top):
                blk = buf[pl.ds(t * L, L)]
                dcl = jnp.maximum(blk, zero_v)
                c_desc, _ = plsc.sort_key_val(dcl, dcl, descending=True)
                lo = jnp.minimum(top, c_desc)
                top_n, _ = plsc.sort_key_val(lo, lo)
                return top_n

            return lax.fori_loop(0, nmerge, body, big_v)

        acc = acc_outer
        for pair in range(L // 2):
            l0, l1 = 2 * pair, 2 * pair + 1
            qq0, qx0, qy0, qz0 = (splat(qqb, l0), splat(qxrb, l0),
                                  splat(qyrb, l0), splat(qzrb, l0))
            qq1, qx1, qy1, qz1 = (splat(qqb, l1), splat(qxrb, l1),
                                  splat(qyrb, l1), splat(qzrb, l1))

            def pass_a(j, carry):
                w0, w1 = carry
                for u in range(4):
                    base = (4 * j + u) * L
                    c2x = candr_v[0, pl.ds(base, L)]
                    c2y = candr_v[1, pl.ds(base, L)]
                    c2z = candr_v[2, pl.ds(base, L)]
                    cc = cc_v[pl.ds(base, L)]
                    i0 = (c2x * qx0 + c2y * qy0) + c2z * qz0
                    d0 = (qq0 - i0) + cc
                    drow_v[pl.ds(base, L)] = d0
                    w0 = jnp.minimum(w0, d0)
                    i1 = (c2x * qx1 + c2y * qy1) + c2z * qz1
                    d1 = (qq1 - i1) + cc
                    drow1_v[pl.ds(base, L)] = d1
                    w1 = jnp.minimum(w1, d1)
                return w0, w1

            w0, w1 = lax.fori_loop(0, NBLK // 4, pass_a, (big_v, big_v))
            thr0, thr1 = thr_of(w0), thr_of(w1)

            def pass_b(j, carry):
                c0, c1 = carry
                for u in range(2):
                    base = (2 * j + u) * L
                    d0 = drow_v[pl.ds(base, L)]
                    m0 = d0 <= thr0
                    plsc.store_compressed(buf_v.at[pl.ds(c0, L)], d0, mask=m0)
                    d1 = drow1_v[pl.ds(base, L)]
                    m1 = d1 <= thr1
                    plsc.store_compressed(buf1_v.at[pl.ds(c1, L)], d1, mask=m1)
                    c0 = c0 + plsc.all_reduce_population_count(m0)[0]
                    c1 = c1 + plsc.all_reduce_population_count(m1)[0]
                return c0, c1

            c0, c1 = lax.fori_loop(0, NBLK // 2, pass_b,
                                   (jnp.int32(0), jnp.int32(0)))
            acc = acc + pass_c(buf_v, c0)
            acc = acc + pass_c(buf1_v, c1)
        return acc

    acc = lax.fori_loop(0, HALF // L, qblk_body,
                        jnp.zeros((L,), dtype=jnp.float32))
    acc_v[...] = acc
    pltpu.sync_copy(acc_v, out_hbm.at[wid])


_knn = functools.partial(
    pl.kernel,
    out_type=jax.ShapeDtypeStruct((NW, L), jnp.float32),
    mesh=plsc.VectorSubcoreMesh(core_axis_name="c", subcore_axis_name="s",
                                num_cores=NC, num_subcores=NS),
    scratch_types=[
        pltpu.VMEM((3, N), jnp.float32),
        pltpu.VMEM((3, N), jnp.float32),
        pltpu.VMEM((N,), jnp.float32),
        pltpu.VMEM((N,), jnp.float32),
        pltpu.VMEM((N,), jnp.float32),
        pltpu.VMEM((N + L,), jnp.float32),
        pltpu.VMEM((N + L,), jnp.float32),
        pltpu.VMEM((L,), jnp.float32),
    ],
    compiler_params=pltpu.CompilerParams(needs_layout_passes=False),
)(_knn_body)


def kernel(seed, gt_s):
    pts = jnp.stack([seed, gt_s])
    pts = pts.transpose(0, 1, 3, 2).reshape(2 * B, 3, N)
    out = _knn(pts)
    per_prob = out.sum(axis=1).reshape(2 * B, 2).sum(axis=1)
    means = (per_prob / (N * K)).reshape(2, B)
    return jnp.mean((means[0] - means[1]) ** 2)


# speedup vs baseline: 3.8127x; 1.2209x over previous
import functools

import jax
import jax.numpy as jnp
from jax import lax
from jax.experimental import pallas as pl
from jax.experimental.pallas import tpu as pltpu
from jax.experimental.pallas import tpu_sc as plsc

NC, NS, L = 2, 16, 16
NW = NC * NS
N = 2048
B = 8
HALF = N // 2
NBLK = N // L
K = 16


def _round_bf16(v):
    u = plsc.bitcast(v, jnp.uint32)
    r = (u + jnp.uint32(0x7FFF) + ((u >> jnp.uint32(16)) & jnp.uint32(1)))
    r = r & jnp.uint32(0xFFFF0000)
    return plsc.bitcast(r, jnp.float32)


BIG = 3.0e38


def _knn_body(pts_hbm, out_hbm, cand_v, candr_v, cc_v, drow_v, buf_v, acc_v):
    wid = lax.axis_index("s") * NC + lax.axis_index("c")
    prob = wid // 2
    half = wid % 2
    pltpu.sync_copy(pts_hbm.at[prob], cand_v)

    big_v = jnp.full((L,), BIG, dtype=jnp.float32)
    zero_v = jnp.zeros((L,), dtype=jnp.float32)

    def pre_body(j, carry):
        base = j * L
        cx = cand_v[0, pl.ds(base, L)]
        cy = cand_v[1, pl.ds(base, L)]
        cz = cand_v[2, pl.ds(base, L)]
        candr_v[0, pl.ds(base, L)] = _round_bf16(cx) * 2.0
        candr_v[1, pl.ds(base, L)] = _round_bf16(cy) * 2.0
        candr_v[2, pl.ds(base, L)] = _round_bf16(cz) * 2.0
        cc_v[pl.ds(base, L)] = (cx * cx + cy * cy) + cz * cz
        return carry

    lax.fori_loop(0, NBLK, pre_body, 0)

    def qblk_body(qb, acc_outer):
        qbase = half * HALF + qb * L
        qxb = cand_v[0, pl.ds(qbase, L)]
        qyb = cand_v[1, pl.ds(qbase, L)]
        qzb = cand_v[2, pl.ds(qbase, L)]
        qqb = (qxb * qxb + qyb * qyb) + qzb * qzb
        qxrb = _round_bf16(qxb)
        qyrb = _round_bf16(qyb)
        qzrb = _round_bf16(qzb)

        def splat(vec, lane):
            return jnp.full((L,), vec[lane], dtype=jnp.float32)

        def thr_of(w):
            ws, _ = plsc.sort_key_val(w, w)
            return jnp.full((L,), ws[L - 1], dtype=jnp.float32)

        def pass_c(r, cnt):
            buf_v[r, pl.ds(cnt, L)] = big_v
            nmerge = lax.div(cnt + (L - 1), jnp.int32(L))

            def body(t, top):
                blk = buf_v[r, pl.ds(t * L, L)]
                dcl = jnp.maximum(blk, zero_v)
                c_desc, _ = plsc.sort_key_val(dcl, dcl, descending=True)
                lo = jnp.minimum(top, c_desc)
                top_n, _ = plsc.sort_key_val(lo, lo)
                return top_n

            return lax.fori_loop(0, nmerge, body, big_v)

        R = 4
        acc = acc_outer
        for quad in range(L // R):
            ls = [R * quad + r for r in range(R)]
            qq = [splat(qqb, l) for l in ls]
            qx = [splat(qxrb, l) for l in ls]
            qy = [splat(qyrb, l) for l in ls]
            qz = [splat(qzrb, l) for l in ls]

            def pass_a(j, carry):
                ws = list(carry)
                for u in range(2):
                    base = (2 * j + u) * L
                    c2x = candr_v[0, pl.ds(base, L)]
                    c2y = candr_v[1, pl.ds(base, L)]
                    c2z = candr_v[2, pl.ds(base, L)]
                    cc = cc_v[pl.ds(base, L)]
                    for r in range(R):
                        i = (c2x * qx[r] + c2y * qy[r]) + c2z * qz[r]
                        d = (qq[r] - i) + cc
                        drow_v[r, pl.ds(base, L)] = d
                        ws[r] = jnp.minimum(ws[r], d)
                return tuple(ws)

            ws = lax.fori_loop(0, NBLK // 2, pass_a, (big_v,) * R)
            thrs = [thr_of(w) for w in ws]

            def pass_b(j, carry):
                cs = list(carry)
                for u in range(2):
                    base = (2 * j + u) * L
                    for r in range(R):
                        d = drow_v[r, pl.ds(base, L)]
                        m = d <= thrs[r]
                        plsc.store_compressed(
                            buf_v.at[r, pl.ds(cs[r], L)], d, mask=m)
                        cs[r] = cs[r] + plsc.all_reduce_population_count(m)[0]
                return tuple(cs)

            cs = lax.fori_loop(0, NBLK // 2, pass_b, (jnp.int32(0),) * R)
            for r in range(R):
                acc = acc + pass_c(r, cs[r])
        return acc

    acc = lax.fori_loop(0, HALF // L, qblk_body,
                        jnp.zeros((L,), dtype=jnp.float32))
    acc_v[...] = acc
    pltpu.sync_copy(acc_v, out_hbm.at[wid])


_knn = functools.partial(
    pl.kernel,
    out_type=jax.ShapeDtypeStruct((NW, L), jnp.float32),
    mesh=plsc.VectorSubcoreMesh(core_axis_name="c", subcore_axis_name="s",
                                num_cores=NC, num_subcores=NS),
    scratch_types=[
        pltpu.VMEM((3, N), jnp.float32),
        pltpu.VMEM((3, N), jnp.float32),
        pltpu.VMEM((N,), jnp.float32),
        pltpu.VMEM((4, N), jnp.float32),
        pltpu.VMEM((4, N + L), jnp.float32),
        pltpu.VMEM((L,), jnp.float32),
    ],
    compiler_params=pltpu.CompilerParams(needs_layout_passes=False),
)(_knn_body)


def kernel(seed, gt_s):
    pts = jnp.stack([seed, gt_s])
    pts = pts.transpose(0, 1, 3, 2).reshape(2 * B, 3, N)
    out = _knn(pts)
    per_prob = out.sum(axis=1).reshape(2 * B, 2).sum(axis=1)
    means = (per_prob / (N * K)).reshape(2, B)
    return jnp.mean((means[0] - means[1]) ** 2)
